# baseline (speedup 1.0000x reference)
import jax
import jax.numpy as jnp
from jax.experimental import pallas as pl
from jax.experimental.pallas import tpu as pltpu

MAP_W = 32
N_NEURONS = 1024
INPUT_DIM = 64
BLOCK_B = 1024
N_TILES = N_NEURONS // 8


def _som_kernel(xsq_ref, xt_ref, w_ref, bmu_ref, qe_ref, wsq_ref):
    @pl.when(pl.program_id(0) == 0)
    def _():
        w0 = w_ref[...]
        wsq_ref[...] = jnp.sum(w0 * w0, axis=1, keepdims=True)

    x = xt_ref[...]
    w2 = w_ref[...] + w_ref[...]
    cross2 = jax.lax.dot_general(
        w2, x, (((1,), (1,)), ((), ())),
        preferred_element_type=jnp.float32)
    x_sq = xsq_ref[0]
    e = (x_sq + wsq_ref[...]) - cross2

    nodes = []
    for r in range(0, N_TILES, 2):
        av = e[8 * r:8 * r + 8, :]
        bv = e[8 * r + 8:8 * r + 16, :]
        take_b = bv < av
        v = jnp.minimum(av, bv)
        i = jnp.where(take_b, jnp.float32(r + 1), jnp.float32(r))
        nodes.append((v, i))
    while len(nodes) > 1:
        nxt = []
        for j in range(0, len(nodes), 2):
            av, ai = nodes[j]
            bv, bi = nodes[j + 1]
            take_b = bv < av
            nxt.append((jnp.minimum(av, bv), jnp.where(take_b, bi, ai)))
        nodes = nxt
    mv, mi = nodes[0]

    sub_iota = jax.lax.broadcasted_iota(
        jnp.int32, (8, BLOCK_B), 0).astype(jnp.float32)
    k8 = mi * 8.0 + sub_iota
    m = jnp.min(mv, axis=0, keepdims=True)
    bmu_f = jnp.min(jnp.where(mv == m, k8, float(N_NEURONS)),
                    axis=0, keepdims=True)
    bmu_ref[...] = bmu_f.astype(jnp.int32)[None]
    qe_ref[...] = jnp.sqrt(jnp.maximum(m, 0.0) + 1e-12)[None]


def kernel(inputs, weights):
    b = inputs.shape[0]
    nb = b // BLOCK_B
    x_sq = jnp.sum(inputs * inputs, axis=1).reshape(nb, 1, BLOCK_B)
    bmu, qe = pl.pallas_call(
        _som_kernel,
        grid=(nb,),
        in_specs=[
            pl.BlockSpec((1, 1, BLOCK_B), lambda i: (i, 0, 0)),
            pl.BlockSpec((BLOCK_B, INPUT_DIM), lambda i: (i, 0)),
            pl.BlockSpec((N_NEURONS, INPUT_DIM), lambda i: (0, 0)),
        ],
        out_specs=[
            pl.BlockSpec((1, 1, BLOCK_B), lambda i: (i, 0, 0)),
            pl.BlockSpec((1, 1, BLOCK_B), lambda i: (i, 0, 0)),
        ],
        out_shape=[
            jax.ShapeDtypeStruct((nb, 1, BLOCK_B), jnp.int32),
            jax.ShapeDtypeStruct((nb, 1, BLOCK_B), jnp.float32),
        ],
        scratch_shapes=[pltpu.VMEM((N_NEURONS, 1), jnp.float32)],
    )(x_sq, inputs, weights)
    flat = bmu.reshape(b)
    coords = jnp.stack([flat // MAP_W, flat % MAP_W], axis=1)
    return coords, qe.reshape(b)


# speedup vs baseline: 1.3187x; 1.3187x over previous
import jax
import jax.numpy as jnp
from jax.experimental import pallas as pl
from jax.experimental.pallas import tpu as pltpu

MAP_W = 32
N_NEURONS = 1024
INPUT_DIM = 64
BLOCK_B = 1024
N_TILES = N_NEURONS // 8


def _som_kernel(xsq_ref, xt_ref, w_ref, bmu_ref, qe_ref, wsq_ref):
    @pl.when(pl.program_id(0) == 0)
    def _():
        w0 = w_ref[...]
        wsq_ref[...] = jnp.sum(w0 * w0, axis=1, keepdims=True)

    xt = xt_ref[...]
    w2 = w_ref[...] + w_ref[...]
    cross2 = jax.lax.dot_general(
        w2, xt, (((1,), (0,)), ((), ())),
        preferred_element_type=jnp.float32)
    x_sq = xsq_ref[0]
    e = (x_sq + wsq_ref[...]) - cross2

    nodes = []
    for r in range(0, N_TILES, 2):
        av = e[8 * r:8 * r + 8, :]
        bv = e[8 * r + 8:8 * r + 16, :]
        take_b = bv < av
        v = jnp.minimum(av, bv)
        i = jnp.where(take_b, jnp.float32(r + 1), jnp.float32(r))
        nodes.append((v, i))
    while len(nodes) > 1:
        nxt = []
        for j in range(0, len(nodes), 2):
            av, ai = nodes[j]
            bv, bi = nodes[j + 1]
            take_b = bv < av
            nxt.append((jnp.minimum(av, bv), jnp.where(take_b, bi, ai)))
        nodes = nxt
    mv, mi = nodes[0]

    sub_iota = jax.lax.broadcasted_iota(
        jnp.int32, (8, BLOCK_B), 0).astype(jnp.float32)
    k8 = mi * 8.0 + sub_iota
    m = jnp.min(mv, axis=0, keepdims=True)
    bmu_f = jnp.min(jnp.where(mv == m, k8, float(N_NEURONS)),
                    axis=0, keepdims=True)
    bmu_ref[...] = bmu_f.astype(jnp.int32)[None]
    qe_ref[...] = jnp.sqrt(jnp.maximum(m, 0.0) + 1e-12)[None]


def kernel(inputs, weights):
    b = inputs.shape[0]
    nb = b // BLOCK_B
    x_sq = jnp.sum(inputs * inputs, axis=1).reshape(nb, 1, BLOCK_B)
    xt = inputs.T
    bmu, qe = pl.pallas_call(
        _som_kernel,
        grid=(nb,),
        in_specs=[
            pl.BlockSpec((1, 1, BLOCK_B), lambda i: (i, 0, 0)),
            pl.BlockSpec((INPUT_DIM, BLOCK_B), lambda i: (0, i)),
            pl.BlockSpec((N_NEURONS, INPUT_DIM), lambda i: (0, 0)),
        ],
        out_specs=[
            pl.BlockSpec((1, 1, BLOCK_B), lambda i: (i, 0, 0)),
            pl.BlockSpec((1, 1, BLOCK_B), lambda i: (i, 0, 0)),
        ],
        out_shape=[
            jax.ShapeDtypeStruct((nb, 1, BLOCK_B), jnp.int32),
            jax.ShapeDtypeStruct((nb, 1, BLOCK_B), jnp.float32),
        ],
        scratch_shapes=[pltpu.VMEM((N_NEURONS, 1), jnp.float32)],
    )(x_sq, xt, weights)
    flat = bmu.reshape(b)
    coords = jnp.stack([flat // MAP_W, flat % MAP_W], axis=1)
    return coords, qe.reshape(b)
